# baseline (speedup 1.0000x reference)
import functools

import jax
import jax.numpy as jnp
from jax import lax
from jax.experimental import pallas as pl
from jax.experimental.pallas import tpu as pltpu
from jax.experimental.pallas import tpu_sc as plsc

_NUM_DIM = 10_000_000
_NUM_ACTIVE = 5_000_000
_NC = 2
_NS = 16
_NW = _NC * _NS
_CHUNK = 8000
_NCHUNK = _NUM_ACTIVE // _CHUNK
_LANES = 16

_mesh = plsc.VectorSubcoreMesh(core_axis_name="c", subcore_axis_name="s")


@functools.partial(
    pl.kernel,
    out_type=jax.ShapeDtypeStruct((_NW, _LANES), jnp.float32),
    mesh=_mesh,
    scratch_types=[
        pltpu.VMEM((_CHUNK,), jnp.int32),
        pltpu.VMEM((_CHUNK,), jnp.float32),
        pltpu.VMEM((_LANES,), jnp.float32),
        pltpu.SemaphoreType.DMA,
    ],
)
def _gather_sq_partials(idx_hbm, table_hbm, x0_hbm, out_hbm,
                        idx_v, rows_v, x0_v, sem):
    wid = lax.axis_index("s") * _NC + lax.axis_index("c")
    pltpu.sync_copy(x0_hbm, x0_v)
    x0 = x0_v[...]

    n_extra = _NCHUNK % _NW
    n_mine = _NCHUNK // _NW + jnp.where(wid < n_extra, 1, 0)

    def chunk_body(g, acc):
        c = wid + g * _NW
        pltpu.sync_copy(idx_hbm.at[pl.ds(c * _CHUNK, _CHUNK)], idx_v)
        pltpu.async_copy(table_hbm.at[idx_v], rows_v, sem).wait()

        def inner(i, acc):
            v = rows_v[pl.ds(i * _LANES, _LANES)]
            d = v - x0
            return acc + d * d

        return lax.fori_loop(0, _CHUNK // _LANES, inner, acc)

    acc = lax.fori_loop(0, n_mine, chunk_body,
                        jnp.zeros((_LANES,), jnp.float32))
    x0_v[...] = acc
    pltpu.sync_copy(x0_v, out_hbm.at[wid])


def kernel(parameters_, active_idx, x_0):
    x0_vec = jnp.full((_LANES,), x_0, dtype=jnp.float32)
    partials = _gather_sq_partials(active_idx, parameters_, x0_vec)
    return -jnp.sum(partials)


# speedup vs baseline: 1.2441x; 1.2441x over previous
import jax
import jax.numpy as jnp
from jax import lax
from jax.experimental import pallas as pl
from jax.experimental.pallas import tpu as pltpu
from jax.experimental.pallas import tpu_sc as plsc

_NUM_DIM = 10_000_000
_NUM_ACTIVE = 5_000_000
_NC = 2
_NS = 16
_NW = _NC * _NS
_CHUNK = 8000
_NCHUNK = _NUM_ACTIVE // _CHUNK
_GMAX = -(-_NCHUNK // _NW)
_LANES = 16
_UNROLL = 4

_mesh = plsc.VectorSubcoreMesh(core_axis_name="c", subcore_axis_name="s")


@pl.kernel(
    out_type=jax.ShapeDtypeStruct((_NW, _LANES), jnp.float32),
    mesh=_mesh,
    scratch_types=[
        pltpu.VMEM((_CHUNK,), jnp.int32),
        pltpu.VMEM((_CHUNK,), jnp.int32),
        pltpu.VMEM((_CHUNK,), jnp.float32),
        pltpu.VMEM((_CHUNK,), jnp.float32),
        pltpu.VMEM((_LANES,), jnp.float32),
        pltpu.SemaphoreType.DMA,
        pltpu.SemaphoreType.DMA,
        pltpu.SemaphoreType.DMA,
        pltpu.SemaphoreType.DMA,
    ],
)
def _gather_sq_partials(idx_hbm, table_hbm, x0_hbm, out_hbm,
                        idx0, idx1, rows0, rows1, stage,
                        si0, si1, sg0, sg1):
    wid = lax.axis_index("s") * _NC + lax.axis_index("c")
    pltpu.sync_copy(x0_hbm, stage)
    x0 = stage[...]

    idx_b = (idx0, idx1)
    rows_b = (rows0, rows1)
    si = (si0, si1)
    sg = (sg0, sg1)

    n_extra = _NCHUNK % _NW
    n_mine = _NCHUNK // _NW + jnp.where(wid < n_extra, 1, 0)

    def cid(g):
        return jnp.minimum(wid + g * _NW, _NCHUNK - 1)

    def start_idx(g):
        b = g % 2
        return pltpu.async_copy(
            idx_hbm.at[pl.ds(cid(g) * _CHUNK, _CHUNK)], idx_b[b], si[b])

    def start_gather(g):
        b = g % 2
        return pltpu.async_copy(table_hbm.at[idx_b[b]], rows_b[b], sg[b])

    def reduce_chunk(rows):
        z = jnp.zeros((_LANES,), jnp.float32)

        def inner(i, accs):
            base = i * (_LANES * _UNROLL)
            out = []
            for u in range(_UNROLL):
                v = rows[pl.ds(base + u * _LANES, _LANES)]
                d = v - x0
                out.append(accs[u] + d * d)
            return tuple(out)

        accs = lax.fori_loop(0, _CHUNK // (_LANES * _UNROLL), inner,
                             (z,) * _UNROLL)
        return (accs[0] + accs[1]) + (accs[2] + accs[3])

    c_i0 = start_idx(0)
    c_i1 = start_idx(1)
    c_i0.wait()
    pend_i = c_i1
    pend_g = start_gather(0)

    acc = jnp.zeros((_LANES,), jnp.float32)
    for g in range(_GMAX):
        pend_g.wait()
        if g + 1 < _GMAX:
            pend_i.wait()
            next_g = start_gather(g + 1)
        if g + 2 < _GMAX:
            pend_i = start_idx(g + 2)
        csum = reduce_chunk(rows_b[g % 2])
        acc = acc + jnp.where(g < n_mine, csum, jnp.zeros_like(csum))
        if g + 1 < _GMAX:
            pend_g = next_g

    stage[...] = acc
    pltpu.sync_copy(stage, out_hbm.at[wid])


def kernel(parameters_, active_idx, x_0):
    x0_vec = jnp.full((_LANES,), x_0, dtype=jnp.float32)
    partials = _gather_sq_partials(active_idx, parameters_, x0_vec)
    return -jnp.sum(partials)
